# baseline (speedup 1.0000x reference)
import jax
import jax.numpy as jnp
from jax import lax
from jax.experimental import pallas as pl
from jax.experimental.pallas import tpu as pltpu
from jax.experimental.pallas import tpu_sc as plsc

N = 10000
E = 320000
D = 128
EPS = 1e-5

NC = 2
NS = 16
NW = NC * NS
CHUNK = 128
NCHB = 8
NBLK = 10
NCH = NCHB * NBLK
EPAD = NW * NCH * CHUNK
C4 = 56
GB = 16
NB0 = 6
NB1 = 17
NCH0 = GB * NB0
NCH1 = GB * NB1
GPAD = NS * (NCH0 + NCH1) * C4
E0 = NS * NCH0 * C4
ROWS_PER_TILE = 640
NPAD = NS * ROWS_PER_TILE
CROWS = NPAD // 128
NSET = 4
OSTEPS = [(k * C4, C4) for k in range(ROWS_PER_TILE // C4)]
OSTEPS.append(((ROWS_PER_TILE // C4) * C4, ROWS_PER_TILE % C4))


def _seg_kernel_body(x_hbm, srcs0, dsts0, srcs1, dsts1, agg_out, acc_s,
                     sg0, sg1, sg2, sg3, ss0, ss1, ss2, ss3):
    cid = lax.axis_index("c")
    sid = lax.axis_index("s")
    base = sid * ROWS_PER_TILE
    semG = [sg0, sg1, sg2, sg3]
    semS = [ss0, ss1, ss2, ss3]

    def _inner(src_v, dst_v, buf0, buf1, buf2, buf3):
        bufs = [buf0, buf1, buf2, buf3]

        def _zrow(i, _):
            def _z16(j, _):
                buf0[i, pl.ds(j * 16, 16)] = jnp.zeros((16,), jnp.float32)
                return 0
            lax.fori_loop(0, D // 16, _z16, 0)
            return 0
        lax.fori_loop(0, C4, _zrow, 0)

        for off, rows in OSTEPS:
            pltpu.sync_copy(buf0.at[pl.ds(0, rows)],
                            acc_s.at[pl.ds(base + off, rows)])

        plsc.subcore_barrier()

        def _run(srcs, dsts, nblk):
            def _blk(bi, _):
                pltpu.sync_copy(srcs.at[sid, pl.ds(bi * GB, GB)], src_v)
                pltpu.sync_copy(dsts.at[sid, pl.ds(bi * GB, GB)], dst_v)
                g = {}
                s = {}
                for j in range(3):
                    g[j] = pltpu.async_copy(x_hbm.at[src_v.at[j]], bufs[j],
                                            semG[j])
                waited = set()
                for j in range(GB):
                    b = j % 4
                    g[j].wait()
                    s[j] = pltpu.async_copy(bufs[b], acc_s.at[dst_v.at[j]],
                                            semS[b], add=True)
                    if j + 3 < GB:
                        if j - 1 >= 0:
                            s[j - 1].wait()
                            waited.add(j - 1)
                        g[j + 3] = pltpu.async_copy(x_hbm.at[src_v.at[j + 3]],
                                                    bufs[(j + 3) % 4],
                                                    semG[(j + 3) % 4])
                for j in range(GB):
                    if j not in waited:
                        s[j].wait()
                return 0
            lax.fori_loop(0, nblk, _blk, 0)

        @pl.when(cid == 0)
        def _():
            _run(srcs0, dsts0, NB0)

        @pl.when(cid == 1)
        def _():
            _run(srcs1, dsts1, NB1)

        plsc.subcore_barrier()

        for off, rows in OSTEPS:
            pltpu.sync_copy(acc_s.at[pl.ds(base + off, rows)],
                            buf0.at[pl.ds(0, rows)])
            pltpu.sync_copy(buf0.at[pl.ds(0, rows)],
                            agg_out.at[cid, pl.ds(base + off, rows)])

    pl.run_scoped(
        _inner,
        src_v=pltpu.VMEM((GB, C4), jnp.int32),
        dst_v=pltpu.VMEM((GB, C4), jnp.int32),
        buf0=pltpu.VMEM((C4, D), jnp.float32),
        buf1=pltpu.VMEM((C4, D), jnp.float32),
        buf2=pltpu.VMEM((C4, D), jnp.float32),
        buf3=pltpu.VMEM((C4, D), jnp.float32),
    )


_seg_sum = pl.kernel(
    _seg_kernel_body,
    out_type=jax.ShapeDtypeStruct((NC, NPAD, D), jnp.float32),
    mesh=plsc.VectorSubcoreMesh(core_axis_name="c", subcore_axis_name="s"),
    scratch_types=[
        pltpu.VMEM_SHARED((NPAD, D), jnp.float32),
        pltpu.SemaphoreType.DMA,
        pltpu.SemaphoreType.DMA,
        pltpu.SemaphoreType.DMA,
        pltpu.SemaphoreType.DMA,
        pltpu.SemaphoreType.DMA,
        pltpu.SemaphoreType.DMA,
        pltpu.SemaphoreType.DMA,
        pltpu.SemaphoreType.DMA,
    ],
)


def _deg_kernel_body(dsts4, cnt_out, sem):
    cid = lax.axis_index("c")
    sid = lax.axis_index("s")
    wid = sid * NC + cid

    def _inner(dst_v, cnt_v):
        ones16 = jnp.ones((16,), jnp.float32)

        def _set(e, _):
            def _zrow(i, _):
                def _z16(j, _):
                    cnt_v[i, pl.ds(j * 16, 16)] = jnp.zeros((16,), jnp.float32)
                    return 0
                lax.fori_loop(0, D // 16, _z16, 0)
                return 0
            lax.fori_loop(0, CROWS, _zrow, 0)

            def _blk(bi, _):
                pltpu.sync_copy(dsts4.at[e, wid, pl.ds(bi * NCHB, NCHB)], dst_v)

                def _chunk(j, _):
                    def _grp(k, _):
                        idx = dst_v[j, pl.ds(k * 16, 16)]
                        row = lax.shift_right_logical(idx, 7)
                        col = lax.bitwise_and(idx, 127)
                        plsc.addupdate_scatter(cnt_v, [row, col], ones16)
                        return 0
                    lax.fori_loop(0, CHUNK // 16, _grp, 0)
                    return 0
                lax.fori_loop(0, NCHB, _chunk, 0)
                return 0
            lax.fori_loop(0, NBLK, _blk, 0)

            pltpu.sync_copy(cnt_v, cnt_out.at[e, wid])
            return 0
        lax.fori_loop(0, NSET, _set, 0)

    pl.run_scoped(
        _inner,
        dst_v=pltpu.VMEM((NCHB, CHUNK), jnp.int32),
        cnt_v=pltpu.VMEM((CROWS, 128), jnp.float32),
    )


_deg = pl.kernel(
    _deg_kernel_body,
    out_type=jax.ShapeDtypeStruct((NSET, NW, CROWS, 128), jnp.float32),
    mesh=plsc.VectorSubcoreMesh(core_axis_name="c", subcore_axis_name="s"),
    scratch_types=[pltpu.SemaphoreType.DMA],
    compiler_params=pltpu.CompilerParams(needs_layout_passes=False),
)


def _cntsum_body(cntp, out):
    out[...] = jnp.sum(cntp[...], axis=1)


_cntsum = pl.pallas_call(
    _cntsum_body,
    out_shape=jax.ShapeDtypeStruct((NSET, CROWS, 128), jnp.float32),
)


def _sage_tc_body(aggp, denom, x, Wl, Wr, b, g, bb, out):
    agg = aggp[0, :N] + aggp[1, :N]
    z = (jnp.dot(agg / denom[...], Wl[...], preferred_element_type=jnp.float32)
         + jnp.dot(x[...], Wr[...], preferred_element_type=jnp.float32)
         + b[...])
    mu = jnp.mean(z, axis=0, keepdims=True)
    var = jnp.mean((z - mu) ** 2, axis=0, keepdims=True)
    h = (z - mu) / jnp.sqrt(var + EPS) * g[...] + bb[...]
    out[...] = jnp.maximum(h, 0.0)


_sage_tc = pl.pallas_call(
    _sage_tc_body,
    out_shape=jax.ShapeDtypeStruct((N, D), jnp.float32),
)


def _heads_body(f, cW1, cb1, cW2, cb2, dW1, db1, dg, db, dW2, db2,
                pred, dom, fmean):
    fv = f[...]
    h1 = jnp.maximum(jnp.dot(fv, cW1[...], preferred_element_type=jnp.float32)
                     + cb1[...], 0.0)
    pred[...] = jnp.dot(h1, cW2[...], preferred_element_type=jnp.float32) + cb2[...]

    zd = jnp.dot(fv, dW1[...], preferred_element_type=jnp.float32) + db1[...]
    mu = jnp.mean(zd, axis=0, keepdims=True)
    var = jnp.mean((zd - mu) ** 2, axis=0, keepdims=True)
    zn = (zd - mu) / jnp.sqrt(var + EPS) * dg[...] + db[...]
    dom[...] = (jnp.dot(jnp.maximum(zn, 0.0), dW2[...],
                        preferred_element_type=jnp.float32) + db2[...])
    fmean[...] = jnp.mean(fv, axis=0, keepdims=True)


_heads = pl.pallas_call(
    _heads_body,
    out_shape=(
        jax.ShapeDtypeStruct((N, 10), jnp.float32),
        jax.ShapeDtypeStruct((N, 2), jnp.float32),
        jax.ShapeDtypeStruct((1, D), jnp.float32),
    ),
)


def _mmd_body(ms, mt, out):
    d = ms[...] - mt[...]
    out[...] = jnp.sum(d * d, keepdims=True).reshape(1, 1)


_mmd = pl.pallas_call(
    _mmd_body,
    out_shape=jax.ShapeDtypeStruct((1, 1), jnp.float32),
)


def _prep_edges(ei):
    src = ei[0]
    dst = ei[1]
    pad = max(EPAD, GPAD) - E
    trash = N + (jnp.arange(pad, dtype=jnp.int32) % (NPAD - N))
    src = jnp.concatenate([src, jnp.zeros((pad,), jnp.int32)])
    dst = jnp.concatenate([dst, trash])
    s0 = src[:E0].reshape(NS, NCH0, C4)
    d0 = dst[:E0].reshape(NS, NCH0, C4)
    s1 = src[E0:GPAD].reshape(NS, NCH1, C4)
    d1 = dst[E0:GPAD].reshape(NS, NCH1, C4)
    return (s0, d0, s1, d1, dst[:EPAD].reshape(NW, NCH, CHUNK))


def _extract(x, eA, eB, den0, den1,
             W1l, W1r, b1, g1, bb1, W2l, W2r, b2, g2, bb2):
    agg0 = _seg_sum(x, eA[0], eA[1], eA[2], eA[3])
    h = _sage_tc(agg0, den0, x, W1l, W1r, b1, g1, bb1)
    agg1 = _seg_sum(h, eB[0], eB[1], eB[2], eB[3])
    return _sage_tc(agg1, den1, h, W2l, W2r, b2, g2, bb2)


def kernel(x_s, x_t, edge_index_s0, edge_index_s1, edge_index_t0, edge_index_t1,
           W1l, W1r, b1, bn1_g, bn1_b, W2l, W2r, b2, bn2_g, bn2_b,
           cls_W1, cls_b1, cls_W2, cls_b2,
           dom_W1, dom_b1, dom_bn_g, dom_bn_b, dom_W2, dom_b2):
    row = lambda v: v.reshape(1, -1)
    b1r, g1r, bb1r = row(b1), row(bn1_g), row(bn1_b)
    b2r, g2r, bb2r = row(b2), row(bn2_g), row(bn2_b)

    edges = [_prep_edges(e) for e in (edge_index_s0, edge_index_s1,
                                      edge_index_t0, edge_index_t1)]
    dsts4 = jnp.stack([e[4] for e in edges])
    cnt_p = _deg(dsts4)
    cnt = _cntsum(cnt_p)
    dens = jnp.maximum(cnt.reshape(NSET, NPAD, 1)[:, :N], 1.0)

    fs = _extract(x_s, edges[0], edges[1], dens[0], dens[1],
                  W1l, W1r, b1r, g1r, bb1r, W2l, W2r, b2r, g2r, bb2r)
    ft = _extract(x_t, edges[2], edges[3], dens[2], dens[3],
                  W1l, W1r, b1r, g1r, bb1r, W2l, W2r, b2r, g2r, bb2r)

    s_pred, s_dom, ms = _heads(fs, cls_W1, row(cls_b1), cls_W2, row(cls_b2),
                               dom_W1, row(dom_b1), row(dom_bn_g),
                               row(dom_bn_b), dom_W2, row(dom_b2))
    t_pred, t_dom, mt = _heads(ft, cls_W1, row(cls_b1), cls_W2, row(cls_b2),
                               dom_W1, row(dom_b1), row(dom_bn_g),
                               row(dom_bn_b), dom_W2, row(dom_b2))
    loss_mmd = _mmd(ms, mt)[0, 0]
    return (s_pred, t_pred, s_dom, t_dom, loss_mmd)


# speedup vs baseline: 1.9317x; 1.9317x over previous
import jax
import jax.numpy as jnp
from jax import lax
from jax.experimental import pallas as pl
from jax.experimental.pallas import tpu as pltpu
from jax.experimental.pallas import tpu_sc as plsc

N = 10000
E = 320000
D = 128
EPS = 1e-5

NC = 2
NS = 16
NW = NC * NS
CHUNK = 128
NCHB = 8
NBLK = 10
NCH = NCHB * NBLK
EPAD = NW * NCH * CHUNK
C4 = 56
NB0 = 12
NB1 = 33
NCH0 = NCHB * NB0
NCH1 = NCHB * NB1
GPAD = NS * (NCH0 + NCH1) * C4
E0 = NS * NCH0 * C4
ROWS_PER_TILE = 640
NPAD = NS * ROWS_PER_TILE
CROWS = NPAD // 128
NSET = 4
OSTEPS = [(k * C4, C4) for k in range(ROWS_PER_TILE // C4)]
OSTEPS.append(((ROWS_PER_TILE // C4) * C4, ROWS_PER_TILE % C4))


def _seg_kernel_body(x_hbm, srcs0, dsts0, srcs1, dsts1, agg_out, acc_s,
                     sg0, sg1, sg2, sg3, ss0, ss1, ss2, ss3):
    cid = lax.axis_index("c")
    sid = lax.axis_index("s")
    base = sid * ROWS_PER_TILE
    semG = [sg0, sg1, sg2, sg3]
    semS = [ss0, ss1, ss2, ss3]

    def _inner(src_v, dst_v, buf0, buf1, buf2, buf3):
        bufs = [buf0, buf1, buf2, buf3]

        def _zrow(i, _):
            def _z16(j, _):
                buf0[i, pl.ds(j * 16, 16)] = jnp.zeros((16,), jnp.float32)
                return 0
            lax.fori_loop(0, D // 16, _z16, 0)
            return 0
        lax.fori_loop(0, C4, _zrow, 0)

        for off, rows in OSTEPS:
            pltpu.sync_copy(buf0.at[pl.ds(0, rows)],
                            acc_s.at[pl.ds(base + off, rows)])

        plsc.subcore_barrier()

        def _run(srcs, dsts, nblk):
            def _blk(bi, _):
                pltpu.sync_copy(srcs.at[sid, pl.ds(bi * NCHB, NCHB)], src_v)
                pltpu.sync_copy(dsts.at[sid, pl.ds(bi * NCHB, NCHB)], dst_v)
                g = {}
                s = {}
                g[0] = pltpu.async_copy(x_hbm.at[src_v.at[0]], bufs[0], semG[0])
                g[1] = pltpu.async_copy(x_hbm.at[src_v.at[1]], bufs[1], semG[1])
                g[2] = pltpu.async_copy(x_hbm.at[src_v.at[2]], bufs[2], semG[2])
                waited = set()
                for j in range(NCHB):
                    b = j % 4
                    g[j].wait()
                    s[j] = pltpu.async_copy(bufs[b], acc_s.at[dst_v.at[j]],
                                            semS[b], add=True)
                    if j + 3 < NCHB:
                        if j - 1 >= 0:
                            s[j - 1].wait()
                            waited.add(j - 1)
                        g[j + 3] = pltpu.async_copy(x_hbm.at[src_v.at[j + 3]],
                                                    bufs[(j + 3) % 4],
                                                    semG[(j + 3) % 4])
                for j in range(NCHB):
                    if j not in waited:
                        s[j].wait()
                return 0
            lax.fori_loop(0, nblk, _blk, 0)

        @pl.when(cid == 0)
        def _():
            _run(srcs0, dsts0, NB0)

        @pl.when(cid == 1)
        def _():
            _run(srcs1, dsts1, NB1)

        plsc.subcore_barrier()

        for off, rows in OSTEPS:
            pltpu.sync_copy(acc_s.at[pl.ds(base + off, rows)],
                            buf0.at[pl.ds(0, rows)])
            pltpu.sync_copy(buf0.at[pl.ds(0, rows)],
                            agg_out.at[cid, pl.ds(base + off, rows)])

    pl.run_scoped(
        _inner,
        src_v=pltpu.VMEM((NCHB, C4), jnp.int32),
        dst_v=pltpu.VMEM((NCHB, C4), jnp.int32),
        buf0=pltpu.VMEM((C4, D), jnp.float32),
        buf1=pltpu.VMEM((C4, D), jnp.float32),
        buf2=pltpu.VMEM((C4, D), jnp.float32),
        buf3=pltpu.VMEM((C4, D), jnp.float32),
    )


_seg_sum = pl.kernel(
    _seg_kernel_body,
    out_type=jax.ShapeDtypeStruct((NC, NPAD, D), jnp.float32),
    mesh=plsc.VectorSubcoreMesh(core_axis_name="c", subcore_axis_name="s"),
    scratch_types=[
        pltpu.VMEM_SHARED((NPAD, D), jnp.float32),
        pltpu.SemaphoreType.DMA,
        pltpu.SemaphoreType.DMA,
        pltpu.SemaphoreType.DMA,
        pltpu.SemaphoreType.DMA,
        pltpu.SemaphoreType.DMA,
        pltpu.SemaphoreType.DMA,
        pltpu.SemaphoreType.DMA,
        pltpu.SemaphoreType.DMA,
    ],
)


def _deg_kernel_body(dsts4, cnt_out, sem):
    cid = lax.axis_index("c")
    sid = lax.axis_index("s")
    wid = sid * NC + cid

    def _inner(dst_v, cnt_v):
        ones16 = jnp.ones((16,), jnp.float32)

        def _set(e, _):
            def _zrow(i, _):
                def _z16(j, _):
                    cnt_v[i, pl.ds(j * 16, 16)] = jnp.zeros((16,), jnp.float32)
                    return 0
                lax.fori_loop(0, D // 16, _z16, 0)
                return 0
            lax.fori_loop(0, CROWS, _zrow, 0)

            def _blk(bi, _):
                pltpu.sync_copy(dsts4.at[e, wid, pl.ds(bi * NCHB, NCHB)], dst_v)

                def _chunk(j, _):
                    def _grp(k, _):
                        idx = dst_v[j, pl.ds(k * 16, 16)]
                        row = lax.shift_right_logical(idx, 7)
                        col = lax.bitwise_and(idx, 127)
                        plsc.addupdate_scatter(cnt_v, [row, col], ones16)
                        return 0
                    lax.fori_loop(0, CHUNK // 16, _grp, 0)
                    return 0
                lax.fori_loop(0, NCHB, _chunk, 0)
                return 0
            lax.fori_loop(0, NBLK, _blk, 0)

            pltpu.sync_copy(cnt_v, cnt_out.at[e, wid])
            return 0
        lax.fori_loop(0, NSET, _set, 0)

    pl.run_scoped(
        _inner,
        dst_v=pltpu.VMEM((NCHB, CHUNK), jnp.int32),
        cnt_v=pltpu.VMEM((CROWS, 128), jnp.float32),
    )


_deg = pl.kernel(
    _deg_kernel_body,
    out_type=jax.ShapeDtypeStruct((NSET, NW, CROWS, 128), jnp.float32),
    mesh=plsc.VectorSubcoreMesh(core_axis_name="c", subcore_axis_name="s"),
    scratch_types=[pltpu.SemaphoreType.DMA],
    compiler_params=pltpu.CompilerParams(needs_layout_passes=False),
)


def _cntsum_body(cntp, out):
    out[...] = jnp.sum(cntp[...], axis=1)


_cntsum = pl.pallas_call(
    _cntsum_body,
    out_shape=jax.ShapeDtypeStruct((NSET, CROWS, 128), jnp.float32),
)


def _sage_tc_body(aggp, denom, x, Wl, Wr, b, g, bb, out):
    agg = aggp[0, :N] + aggp[1, :N]
    z = (jnp.dot(agg / denom[...], Wl[...], preferred_element_type=jnp.float32)
         + jnp.dot(x[...], Wr[...], preferred_element_type=jnp.float32)
         + b[...])
    mu = jnp.mean(z, axis=0, keepdims=True)
    var = jnp.mean((z - mu) ** 2, axis=0, keepdims=True)
    h = (z - mu) / jnp.sqrt(var + EPS) * g[...] + bb[...]
    out[...] = jnp.maximum(h, 0.0)


_sage_tc = pl.pallas_call(
    _sage_tc_body,
    out_shape=jax.ShapeDtypeStruct((N, D), jnp.float32),
)


def _heads_body(f, cW1, cb1, cW2, cb2, dW1, db1, dg, db, dW2, db2,
                pred, dom, fmean):
    fv = f[...]
    h1 = jnp.maximum(jnp.dot(fv, cW1[...], preferred_element_type=jnp.float32)
                     + cb1[...], 0.0)
    pred[...] = jnp.dot(h1, cW2[...], preferred_element_type=jnp.float32) + cb2[...]

    zd = jnp.dot(fv, dW1[...], preferred_element_type=jnp.float32) + db1[...]
    mu = jnp.mean(zd, axis=0, keepdims=True)
    var = jnp.mean((zd - mu) ** 2, axis=0, keepdims=True)
    zn = (zd - mu) / jnp.sqrt(var + EPS) * dg[...] + db[...]
    dom[...] = (jnp.dot(jnp.maximum(zn, 0.0), dW2[...],
                        preferred_element_type=jnp.float32) + db2[...])
    fmean[...] = jnp.mean(fv, axis=0, keepdims=True)


_heads = pl.pallas_call(
    _heads_body,
    out_shape=(
        jax.ShapeDtypeStruct((N, 10), jnp.float32),
        jax.ShapeDtypeStruct((N, 2), jnp.float32),
        jax.ShapeDtypeStruct((1, D), jnp.float32),
    ),
)


def _mmd_body(ms, mt, out):
    d = ms[...] - mt[...]
    out[...] = jnp.sum(d * d, keepdims=True).reshape(1, 1)


_mmd = pl.pallas_call(
    _mmd_body,
    out_shape=jax.ShapeDtypeStruct((1, 1), jnp.float32),
)


def _prep_edges(ei):
    src = ei[0]
    dst = ei[1]
    pad = max(EPAD, GPAD) - E
    trash = N + (jnp.arange(pad, dtype=jnp.int32) % (NPAD - N))
    src = jnp.concatenate([src, jnp.zeros((pad,), jnp.int32)])
    dst = jnp.concatenate([dst, trash])
    s0 = src[:E0].reshape(NS, NCH0, C4)
    d0 = dst[:E0].reshape(NS, NCH0, C4)
    s1 = src[E0:GPAD].reshape(NS, NCH1, C4)
    d1 = dst[E0:GPAD].reshape(NS, NCH1, C4)
    return (s0, d0, s1, d1, dst[:EPAD].reshape(NW, NCH, CHUNK))


def _extract(x, eA, eB, den0, den1,
             W1l, W1r, b1, g1, bb1, W2l, W2r, b2, g2, bb2):
    agg0 = _seg_sum(x, eA[0], eA[1], eA[2], eA[3])
    h = _sage_tc(agg0, den0, x, W1l, W1r, b1, g1, bb1)
    agg1 = _seg_sum(h, eB[0], eB[1], eB[2], eB[3])
    return _sage_tc(agg1, den1, h, W2l, W2r, b2, g2, bb2)


def kernel(x_s, x_t, edge_index_s0, edge_index_s1, edge_index_t0, edge_index_t1,
           W1l, W1r, b1, bn1_g, bn1_b, W2l, W2r, b2, bn2_g, bn2_b,
           cls_W1, cls_b1, cls_W2, cls_b2,
           dom_W1, dom_b1, dom_bn_g, dom_bn_b, dom_W2, dom_b2):
    row = lambda v: v.reshape(1, -1)
    b1r, g1r, bb1r = row(b1), row(bn1_g), row(bn1_b)
    b2r, g2r, bb2r = row(b2), row(bn2_g), row(bn2_b)

    edges = [_prep_edges(e) for e in (edge_index_s0, edge_index_s1,
                                      edge_index_t0, edge_index_t1)]
    dsts4 = jnp.stack([e[4] for e in edges])
    cnt_p = _deg(dsts4)
    cnt = _cntsum(cnt_p)
    dens = jnp.maximum(cnt.reshape(NSET, NPAD, 1)[:, :N], 1.0)

    fs = _extract(x_s, edges[0], edges[1], dens[0], dens[1],
                  W1l, W1r, b1r, g1r, bb1r, W2l, W2r, b2r, g2r, bb2r)
    ft = _extract(x_t, edges[2], edges[3], dens[2], dens[3],
                  W1l, W1r, b1r, g1r, bb1r, W2l, W2r, b2r, g2r, bb2r)

    s_pred, s_dom, ms = _heads(fs, cls_W1, row(cls_b1), cls_W2, row(cls_b2),
                               dom_W1, row(dom_b1), row(dom_bn_g),
                               row(dom_bn_b), dom_W2, row(dom_b2))
    t_pred, t_dom, mt = _heads(ft, cls_W1, row(cls_b1), cls_W2, row(cls_b2),
                               dom_W1, row(dom_b1), row(dom_bn_g),
                               row(dom_bn_b), dom_W2, row(dom_b2))
    loss_mmd = _mmd(ms, mt)[0, 0]
    return (s_pred, t_pred, s_dom, t_dom, loss_mmd)


# speedup vs baseline: 2.2671x; 1.1736x over previous
import jax
import jax.numpy as jnp
from jax import lax
from jax.experimental import pallas as pl
from jax.experimental.pallas import tpu as pltpu
from jax.experimental.pallas import tpu_sc as plsc

N = 10000
E = 320000
D = 128
EPS = 1e-5

NC = 2
NS = 16
NW = NC * NS
CHUNK = 128
NCHB = 8
NBLK = 10
NCH = NCHB * NBLK
EPAD = NW * NCH * CHUNK
C4 = 56
NB0 = 22
NB1 = 23
NCH0 = NCHB * NB0
NCH1 = NCHB * NB1
GPAD = NS * (NCH0 + NCH1) * C4
E0 = NS * NCH0 * C4
ROWS_PER_TILE = 640
NPAD = NS * ROWS_PER_TILE
CROWS = NPAD // 128
NSET = 4
OSTEPS = [(k * C4, C4) for k in range(ROWS_PER_TILE // C4)]
OSTEPS.append(((ROWS_PER_TILE // C4) * C4, ROWS_PER_TILE % C4))


def _seg_kernel_body(x_hbm, srcs0, dsts0, srcs1, dsts1, agg_out, acc_s,
                     sg0, sg1, sg2, sg3, ss0, ss1, ss2, ss3):
    cid = lax.axis_index("c")
    sid = lax.axis_index("s")
    base = sid * ROWS_PER_TILE
    semG = [sg0, sg1, sg2, sg3]
    semS = [ss0, ss1, ss2, ss3]

    def _inner(src_v, dst_v, buf0, buf1, buf2, buf3):
        bufs = [buf0, buf1, buf2, buf3]

        def _zrow(i, _):
            def _z16(j, _):
                buf0[i, pl.ds(j * 16, 16)] = jnp.zeros((16,), jnp.float32)
                return 0
            lax.fori_loop(0, D // 16, _z16, 0)
            return 0
        lax.fori_loop(0, C4, _zrow, 0)

        for off, rows in OSTEPS:
            pltpu.sync_copy(buf0.at[pl.ds(0, rows)],
                            acc_s.at[pl.ds(base + off, rows)])

        plsc.subcore_barrier()

        def _run(srcs, dsts, nblk):
            def _blk(bi, _):
                pltpu.sync_copy(srcs.at[sid, pl.ds(bi * NCHB, NCHB)], src_v)
                pltpu.sync_copy(dsts.at[sid, pl.ds(bi * NCHB, NCHB)], dst_v)
                g = {}
                s = {}
                g[0] = pltpu.async_copy(x_hbm.at[src_v.at[0]], bufs[0], semG[0])
                g[1] = pltpu.async_copy(x_hbm.at[src_v.at[1]], bufs[1], semG[1])
                g[2] = pltpu.async_copy(x_hbm.at[src_v.at[2]], bufs[2], semG[2])
                waited = set()
                for j in range(NCHB):
                    b = j % 4
                    g[j].wait()
                    s[j] = pltpu.async_copy(bufs[b], acc_s.at[dst_v.at[j]],
                                            semS[b], add=True)
                    if j + 3 < NCHB:
                        if j - 1 >= 0:
                            s[j - 1].wait()
                            waited.add(j - 1)
                        g[j + 3] = pltpu.async_copy(x_hbm.at[src_v.at[j + 3]],
                                                    bufs[(j + 3) % 4],
                                                    semG[(j + 3) % 4])
                for j in range(NCHB):
                    if j not in waited:
                        s[j].wait()
                return 0
            lax.fori_loop(0, nblk, _blk, 0)

        @pl.when(cid == 0)
        def _():
            _run(srcs0, dsts0, NB0)

        @pl.when(cid == 1)
        def _():
            _run(srcs1, dsts1, NB1)

        plsc.subcore_barrier()

        for off, rows in OSTEPS:
            pltpu.sync_copy(acc_s.at[pl.ds(base + off, rows)],
                            buf0.at[pl.ds(0, rows)])
            pltpu.sync_copy(buf0.at[pl.ds(0, rows)],
                            agg_out.at[cid, pl.ds(base + off, rows)])

    pl.run_scoped(
        _inner,
        src_v=pltpu.VMEM((NCHB, C4), jnp.int32),
        dst_v=pltpu.VMEM((NCHB, C4), jnp.int32),
        buf0=pltpu.VMEM((C4, D), jnp.float32),
        buf1=pltpu.VMEM((C4, D), jnp.float32),
        buf2=pltpu.VMEM((C4, D), jnp.float32),
        buf3=pltpu.VMEM((C4, D), jnp.float32),
    )


_seg_sum = pl.kernel(
    _seg_kernel_body,
    out_type=jax.ShapeDtypeStruct((NC, NPAD, D), jnp.float32),
    mesh=plsc.VectorSubcoreMesh(core_axis_name="c", subcore_axis_name="s"),
    scratch_types=[
        pltpu.VMEM_SHARED((NPAD, D), jnp.float32),
        pltpu.SemaphoreType.DMA,
        pltpu.SemaphoreType.DMA,
        pltpu.SemaphoreType.DMA,
        pltpu.SemaphoreType.DMA,
        pltpu.SemaphoreType.DMA,
        pltpu.SemaphoreType.DMA,
        pltpu.SemaphoreType.DMA,
        pltpu.SemaphoreType.DMA,
    ],
)


def _deg_kernel_body(dsts4, cnt_out, sem):
    cid = lax.axis_index("c")
    sid = lax.axis_index("s")
    wid = sid * NC + cid

    def _inner(dst_v, cnt_v):
        ones16 = jnp.ones((16,), jnp.float32)

        def _set(e, _):
            def _zrow(i, _):
                def _z16(j, _):
                    cnt_v[i, pl.ds(j * 16, 16)] = jnp.zeros((16,), jnp.float32)
                    return 0
                lax.fori_loop(0, D // 16, _z16, 0)
                return 0
            lax.fori_loop(0, CROWS, _zrow, 0)

            def _blk(bi, _):
                pltpu.sync_copy(dsts4.at[e, wid, pl.ds(bi * NCHB, NCHB)], dst_v)

                def _chunk(j, _):
                    def _grp(k, _):
                        idx = dst_v[j, pl.ds(k * 16, 16)]
                        row = lax.shift_right_logical(idx, 7)
                        col = lax.bitwise_and(idx, 127)
                        plsc.addupdate_scatter(cnt_v, [row, col], ones16)
                        return 0
                    lax.fori_loop(0, CHUNK // 16, _grp, 0)
                    return 0
                lax.fori_loop(0, NCHB, _chunk, 0)
                return 0
            lax.fori_loop(0, NBLK, _blk, 0)

            pltpu.sync_copy(cnt_v, cnt_out.at[e, wid])
            return 0
        lax.fori_loop(0, NSET, _set, 0)

    pl.run_scoped(
        _inner,
        dst_v=pltpu.VMEM((NCHB, CHUNK), jnp.int32),
        cnt_v=pltpu.VMEM((CROWS, 128), jnp.float32),
    )


_deg = pl.kernel(
    _deg_kernel_body,
    out_type=jax.ShapeDtypeStruct((NSET, NW, CROWS, 128), jnp.float32),
    mesh=plsc.VectorSubcoreMesh(core_axis_name="c", subcore_axis_name="s"),
    scratch_types=[pltpu.SemaphoreType.DMA],
    compiler_params=pltpu.CompilerParams(needs_layout_passes=False),
)


def _cntsum_body(cntp, out):
    out[...] = jnp.sum(cntp[...], axis=1)


_cntsum = pl.pallas_call(
    _cntsum_body,
    out_shape=jax.ShapeDtypeStruct((NSET, CROWS, 128), jnp.float32),
)


def _sage_tc_body(aggp, denom, x, Wl, Wr, b, g, bb, out):
    agg = aggp[0, :N] + aggp[1, :N]
    z = (jnp.dot(agg / denom[...], Wl[...], preferred_element_type=jnp.float32)
         + jnp.dot(x[...], Wr[...], preferred_element_type=jnp.float32)
         + b[...])
    mu = jnp.mean(z, axis=0, keepdims=True)
    var = jnp.mean((z - mu) ** 2, axis=0, keepdims=True)
    h = (z - mu) / jnp.sqrt(var + EPS) * g[...] + bb[...]
    out[...] = jnp.maximum(h, 0.0)


_sage_tc = pl.pallas_call(
    _sage_tc_body,
    out_shape=jax.ShapeDtypeStruct((N, D), jnp.float32),
)


def _heads_body(f, cW1, cb1, cW2, cb2, dW1, db1, dg, db, dW2, db2,
                pred, dom, fmean):
    fv = f[...]
    h1 = jnp.maximum(jnp.dot(fv, cW1[...], preferred_element_type=jnp.float32)
                     + cb1[...], 0.0)
    pred[...] = jnp.dot(h1, cW2[...], preferred_element_type=jnp.float32) + cb2[...]

    zd = jnp.dot(fv, dW1[...], preferred_element_type=jnp.float32) + db1[...]
    mu = jnp.mean(zd, axis=0, keepdims=True)
    var = jnp.mean((zd - mu) ** 2, axis=0, keepdims=True)
    zn = (zd - mu) / jnp.sqrt(var + EPS) * dg[...] + db[...]
    dom[...] = (jnp.dot(jnp.maximum(zn, 0.0), dW2[...],
                        preferred_element_type=jnp.float32) + db2[...])
    fmean[...] = jnp.mean(fv, axis=0, keepdims=True)


_heads = pl.pallas_call(
    _heads_body,
    out_shape=(
        jax.ShapeDtypeStruct((N, 10), jnp.float32),
        jax.ShapeDtypeStruct((N, 2), jnp.float32),
        jax.ShapeDtypeStruct((1, D), jnp.float32),
    ),
)


def _mmd_body(ms, mt, out):
    d = ms[...] - mt[...]
    out[...] = jnp.sum(d * d, keepdims=True).reshape(1, 1)


_mmd = pl.pallas_call(
    _mmd_body,
    out_shape=jax.ShapeDtypeStruct((1, 1), jnp.float32),
)


def _prep_edges(ei):
    src = ei[0]
    dst = ei[1]
    pad = EPAD - E
    src = jnp.concatenate([src, jnp.zeros((pad,), jnp.int32)])
    dst = jnp.concatenate([dst, jnp.full((pad,), N, jnp.int32)])
    s0 = src[:E0].reshape(NS, NCH0, C4)
    d0 = dst[:E0].reshape(NS, NCH0, C4)
    s1 = src[E0:GPAD].reshape(NS, NCH1, C4)
    d1 = dst[E0:GPAD].reshape(NS, NCH1, C4)
    return (s0, d0, s1, d1, dst.reshape(NW, NCH, CHUNK))


def _extract(x, eA, eB, den0, den1,
             W1l, W1r, b1, g1, bb1, W2l, W2r, b2, g2, bb2):
    agg0 = _seg_sum(x, eA[0], eA[1], eA[2], eA[3])
    h = _sage_tc(agg0, den0, x, W1l, W1r, b1, g1, bb1)
    agg1 = _seg_sum(h, eB[0], eB[1], eB[2], eB[3])
    return _sage_tc(agg1, den1, h, W2l, W2r, b2, g2, bb2)


def kernel(x_s, x_t, edge_index_s0, edge_index_s1, edge_index_t0, edge_index_t1,
           W1l, W1r, b1, bn1_g, bn1_b, W2l, W2r, b2, bn2_g, bn2_b,
           cls_W1, cls_b1, cls_W2, cls_b2,
           dom_W1, dom_b1, dom_bn_g, dom_bn_b, dom_W2, dom_b2):
    row = lambda v: v.reshape(1, -1)
    b1r, g1r, bb1r = row(b1), row(bn1_g), row(bn1_b)
    b2r, g2r, bb2r = row(b2), row(bn2_g), row(bn2_b)

    edges = [_prep_edges(e) for e in (edge_index_s0, edge_index_s1,
                                      edge_index_t0, edge_index_t1)]
    dsts4 = jnp.stack([e[4] for e in edges])
    cnt_p = _deg(dsts4)
    cnt = _cntsum(cnt_p)
    dens = jnp.maximum(cnt.reshape(NSET, NPAD, 1)[:, :N], 1.0)

    fs = _extract(x_s, edges[0], edges[1], dens[0], dens[1],
                  W1l, W1r, b1r, g1r, bb1r, W2l, W2r, b2r, g2r, bb2r)
    ft = _extract(x_t, edges[2], edges[3], dens[2], dens[3],
                  W1l, W1r, b1r, g1r, bb1r, W2l, W2r, b2r, g2r, bb2r)

    s_pred, s_dom, ms = _heads(fs, cls_W1, row(cls_b1), cls_W2, row(cls_b2),
                               dom_W1, row(dom_b1), row(dom_bn_g),
                               row(dom_bn_b), dom_W2, row(dom_b2))
    t_pred, t_dom, mt = _heads(ft, cls_W1, row(cls_b1), cls_W2, row(cls_b2),
                               dom_W1, row(dom_b1), row(dom_bn_g),
                               row(dom_bn_b), dom_W2, row(dom_b2))
    loss_mmd = _mmd(ms, mt)[0, 0]
    return (s_pred, t_pred, s_dom, t_dom, loss_mmd)
